# baseline (speedup 1.0000x reference)
import functools

import jax
import jax.numpy as jnp
from jax import lax
from jax.experimental import pallas as pl
from jax.experimental.pallas import tpu as pltpu
from jax.experimental.pallas import tpu_sc as plsc

_NC = 2
_NS = 16
_NW = _NC * _NS

_CHUNK = 32


def _fuse_body(tok_ref, wt_ref, b_ref, f_ref):
    f_ref[...] = (
        jnp.dot(tok_ref[...], wt_ref[...], preferred_element_type=jnp.float32)
        + b_ref[...]
    )


def _fused_table(tok_table, W, b):
    V, D = W.shape
    return pl.pallas_call(
        _fuse_body,
        out_shape=jax.ShapeDtypeStruct((tok_table.shape[0], V), jnp.float32),
    )(tok_table, W.T, b.reshape(1, V))


def _make_gather(V, D, B):
    b_per_w = B // _NW
    n_chunks = b_per_w // _CHUNK
    mesh = plsc.VectorSubcoreMesh(core_axis_name="c", subcore_axis_name="s")

    @functools.partial(
        pl.kernel,
        out_type=jax.ShapeDtypeStruct((B, D), jnp.float32),
        mesh=mesh,
        scratch_types=[
            pltpu.VMEM((b_per_w,), jnp.int32),
            pltpu.VMEM((_CHUNK, D), jnp.float32),
            pltpu.SemaphoreType.DMA,
        ],
        compiler_params=pltpu.CompilerParams(use_tc_tiling_on_sc=False),
    )
    def gather_kernel(table_hbm, idx_hbm, out_hbm, idx_v, rows_v, sem):
        wid = lax.axis_index("s") * _NC + lax.axis_index("c")
        base = wid * b_per_w
        pltpu.sync_copy(idx_hbm.at[pl.ds(base, b_per_w)], idx_v)

        def body(j, carry):
            off = j * _CHUNK
            pltpu.async_copy(
                table_hbm.at[idx_v.at[pl.ds(off, _CHUNK)]], rows_v, sem
            ).wait()
            pltpu.sync_copy(rows_v, out_hbm.at[pl.ds(base + off, _CHUNK)])
            return carry

        lax.fori_loop(0, n_chunks, body, 0)

    return gather_kernel


def kernel(idx, tok_table, pos_table, W, b):
    del pos_table
    Bn, Tn = idx.shape
    V = W.shape[0]
    table = _fused_table(tok_table, W, b)
    flat_idx = idx.reshape(-1).astype(jnp.int32)
    out = _make_gather(tok_table.shape[0], V, Bn * Tn)(table, flat_idx)
    return out.reshape(Bn, Tn, V)


# speedup vs baseline: 1.0613x; 1.0613x over previous
import functools

import jax
import jax.numpy as jnp
from jax import lax
from jax.experimental import pallas as pl
from jax.experimental.pallas import tpu as pltpu
from jax.experimental.pallas import tpu_sc as plsc

_NC = 2
_NS = 16
_NW = _NC * _NS

_CHUNK = 16
_NBUF = 4


def _fuse_body(tok_ref, wt_ref, b_ref, f_ref):
    f_ref[...] = (
        jnp.dot(tok_ref[...], wt_ref[...], preferred_element_type=jnp.float32)
        + b_ref[...]
    )


def _fused_table(tok_table, W, b):
    V, D = W.shape
    return pl.pallas_call(
        _fuse_body,
        out_shape=jax.ShapeDtypeStruct((tok_table.shape[0], V), jnp.float32),
    )(tok_table, W.T, b.reshape(1, V))


def _make_gather(V, D, B):
    b_per_w = B // _NW
    n_chunks = b_per_w // _CHUNK
    n_outer = n_chunks // _NBUF
    assert n_chunks % _NBUF == 0 and n_outer >= 2
    mesh = plsc.VectorSubcoreMesh(core_axis_name="c", subcore_axis_name="s")

    @functools.partial(
        pl.kernel,
        out_type=jax.ShapeDtypeStruct((B, D), jnp.float32),
        mesh=mesh,
        scratch_types=[
            pltpu.VMEM((b_per_w,), jnp.int32),
            pltpu.VMEM((_NBUF, _CHUNK, D), jnp.float32),
            pltpu.SemaphoreType.DMA((_NBUF,)),
            pltpu.SemaphoreType.DMA((_NBUF,)),
        ],
        compiler_params=pltpu.CompilerParams(use_tc_tiling_on_sc=False),
    )
    def gather_kernel(table_hbm, idx_hbm, out_hbm, idx_v, rows_v, gsem, osem):
        wid = lax.axis_index("s") * _NC + lax.axis_index("c")
        base = wid * b_per_w
        pltpu.sync_copy(idx_hbm.at[pl.ds(base, b_per_w)], idx_v)

        def start_gather(j, slot):
            pltpu.async_copy(
                table_hbm.at[idx_v.at[pl.ds(j * _CHUNK, _CHUNK)]],
                rows_v.at[slot],
                gsem.at[slot],
            )

        def start_store(j, slot):
            pltpu.async_copy(
                rows_v.at[slot],
                out_hbm.at[pl.ds(base + j * _CHUNK, _CHUNK)],
                osem.at[slot],
            )

        def wait_gather(j, slot):
            pltpu.make_async_copy(
                table_hbm.at[idx_v.at[pl.ds(j * _CHUNK, _CHUNK)]],
                rows_v.at[slot],
                gsem.at[slot],
            ).wait()

        def wait_store(j, slot):
            pltpu.make_async_copy(
                rows_v.at[slot],
                out_hbm.at[pl.ds(base + j * _CHUNK, _CHUNK)],
                osem.at[slot],
            ).wait()

        for s in range(_NBUF - 1):
            start_gather(s, s)

        def step(g, s):
            j = g * _NBUF + s
            wait_gather(j, s)
            start_store(j, s)
            nxt = (s + _NBUF - 1) % _NBUF
            if s == 0:
                @pl.when(g > 0)
                def _():
                    wait_store(j - 1, nxt)
                start_gather(j + _NBUF - 1, nxt)
            else:
                wait_store(j - 1, nxt)

                @pl.when(g < n_outer - 1)
                def _():
                    start_gather(j + _NBUF - 1, nxt)

        def body(g, carry):
            for s in range(_NBUF):
                step(g, s)
            return carry

        lax.fori_loop(0, n_outer, body, 0)
        wait_store(n_chunks - 1, (n_chunks - 1) % _NBUF)

    return gather_kernel


def kernel(idx, tok_table, pos_table, W, b):
    del pos_table
    Bn, Tn = idx.shape
    V = W.shape[0]
    table = _fused_table(tok_table, W, b)
    flat_idx = idx.reshape(-1).astype(jnp.int32)
    out = _make_gather(tok_table.shape[0], V, Bn * Tn)(table, flat_idx)
    return out.reshape(Bn, Tn, V)


# speedup vs baseline: 1.6055x; 1.5128x over previous
import functools

import jax
import jax.numpy as jnp
from jax import lax
from jax.experimental import pallas as pl
from jax.experimental.pallas import tpu as pltpu
from jax.experimental.pallas import tpu_sc as plsc

_NC = 2
_NS = 16
_NW = _NC * _NS

_CHUNK = 16
_NBUF = 4

_DPAD = 1024
_DA = 896
_DB = _DPAD - _DA

_MBLK = 4096


def _fuse_body(tok_ref, wt_ref, b_ref, fa_ref, fb_ref):
    f = (
        jnp.dot(tok_ref[...], wt_ref[...], preferred_element_type=jnp.float32)
        + b_ref[...]
    )
    fa_ref[...] = f[:, :_DA]
    fb_ref[...] = f[:, _DA:]


def _fused_table(tok_table, W, b):
    V = W.shape[0]
    wt = jnp.pad(W.T, ((0, 0), (0, _DPAD - V)))
    bp = jnp.pad(b.reshape(1, V), ((0, 0), (0, _DPAD - V)))
    nrows = tok_table.shape[0]
    return pl.pallas_call(
        _fuse_body,
        out_shape=[
            jax.ShapeDtypeStruct((nrows, _DA), jnp.float32),
            jax.ShapeDtypeStruct((nrows, _DB), jnp.float32),
        ],
    )(tok_table, wt, bp)


def _make_gather(D, B):
    b_per_w = B // _NW
    n_chunks = b_per_w // _CHUNK
    n_outer = n_chunks // _NBUF
    assert n_chunks % _NBUF == 0 and n_outer >= 2
    mesh = plsc.VectorSubcoreMesh(core_axis_name="c", subcore_axis_name="s")

    @functools.partial(
        pl.kernel,
        out_type=[
            jax.ShapeDtypeStruct((B, D), jnp.float32),
            jax.ShapeDtypeStruct((B, _DB), jnp.float32),
        ],
        mesh=mesh,
        scratch_types=[
            pltpu.VMEM((b_per_w,), jnp.int32),
            pltpu.VMEM((_NBUF, _CHUNK, _DA), jnp.float32),
            pltpu.VMEM((_NBUF, _CHUNK, _DB), jnp.float32),
            pltpu.SemaphoreType.DMA((_NBUF,)),
            pltpu.SemaphoreType.DMA((_NBUF,)),
            pltpu.SemaphoreType.DMA((_NBUF,)),
            pltpu.SemaphoreType.DMA((_NBUF,)),
        ],
    )
    def gather_kernel(fa_hbm, fb_hbm, idx_hbm, out_hbm, outb_hbm,
                      idx_v, ra_v, rb_v, gasem, gbsem, oasem, obsem):
        wid = lax.axis_index("s") * _NC + lax.axis_index("c")
        base = wid * b_per_w
        pltpu.sync_copy(idx_hbm.at[pl.ds(base, b_per_w)], idx_v)

        def ga_copy(j, slot):
            return pltpu.make_async_copy(
                fa_hbm.at[idx_v.at[pl.ds(j * _CHUNK, _CHUNK)]],
                ra_v.at[slot],
                gasem.at[slot],
            )

        def gb_copy(j, slot):
            return pltpu.make_async_copy(
                fb_hbm.at[idx_v.at[pl.ds(j * _CHUNK, _CHUNK)]],
                rb_v.at[slot],
                gbsem.at[slot],
            )

        def oa_copy(j, slot):
            return pltpu.make_async_copy(
                ra_v.at[slot],
                out_hbm.at[pl.ds(base + j * _CHUNK, _CHUNK), pl.ds(0, _DA)],
                oasem.at[slot],
            )

        def ob_copy(j, slot):
            return pltpu.make_async_copy(
                rb_v.at[slot],
                outb_hbm.at[pl.ds(base + j * _CHUNK, _CHUNK)],
                obsem.at[slot],
            )

        def start_gathers(j, slot):
            ga_copy(j, slot).start()
            gb_copy(j, slot).start()

        def start_stores(j, slot):
            oa_copy(j, slot).start()
            ob_copy(j, slot).start()

        def wait_gathers(j, slot):
            ga_copy(j, slot).wait()
            gb_copy(j, slot).wait()

        def wait_stores(j, slot):
            oa_copy(j, slot).wait()
            ob_copy(j, slot).wait()

        for s in range(_NBUF - 1):
            start_gathers(s, s)

        def step(g, s):
            j = g * _NBUF + s
            wait_gathers(j, s)
            start_stores(j, s)
            nxt = (s + _NBUF - 1) % _NBUF
            if s == 0:
                @pl.when(g > 0)
                def _():
                    wait_stores(j - 1, nxt)
                start_gathers(j + _NBUF - 1, nxt)
            else:
                wait_stores(j - 1, nxt)

                @pl.when(g < n_outer - 1)
                def _():
                    start_gathers(j + _NBUF - 1, nxt)

        def body(g, carry):
            for s in range(_NBUF):
                step(g, s)
            return carry

        lax.fori_loop(0, n_outer, body, 0)
        wait_stores(n_chunks - 1, (n_chunks - 1) % _NBUF)

    return gather_kernel


def _merge_body(outb_hbm, outdon_hbm, out_hbm, vbuf, tbuf, isem, osem):
    i = pl.program_id(0)
    del outdon_hbm
    pltpu.make_async_copy(
        outb_hbm.at[pl.ds(i * _MBLK, _MBLK)], vbuf, isem
    ).start()
    pltpu.make_async_copy(
        outb_hbm.at[pl.ds(i * _MBLK, _MBLK)], vbuf, isem
    ).wait()
    tbuf[...] = vbuf[:, : tbuf.shape[1]]
    pltpu.make_async_copy(
        tbuf,
        out_hbm.at[pl.ds(i * _MBLK, _MBLK), pl.ds(_DA, tbuf.shape[1])],
        osem,
    ).start()
    pltpu.make_async_copy(
        tbuf,
        out_hbm.at[pl.ds(i * _MBLK, _MBLK), pl.ds(_DA, tbuf.shape[1])],
        osem,
    ).wait()


def _merge_tail(outb, out1):
    B, D = out1.shape
    tail = D - _DA
    return pl.pallas_call(
        _merge_body,
        grid=(B // _MBLK,),
        in_specs=[
            pl.BlockSpec(memory_space=pl.ANY),
            pl.BlockSpec(memory_space=pl.ANY),
        ],
        out_specs=pl.BlockSpec(memory_space=pl.ANY),
        out_shape=jax.ShapeDtypeStruct((B, D), jnp.float32),
        scratch_shapes=[
            pltpu.VMEM((_MBLK, _DB), jnp.float32),
            pltpu.VMEM((_MBLK, tail), jnp.float32),
            pltpu.SemaphoreType.DMA,
            pltpu.SemaphoreType.DMA,
        ],
        input_output_aliases={1: 0},
    )(outb, out1)


def kernel(idx, tok_table, pos_table, W, b):
    del pos_table
    Bn, Tn = idx.shape
    V = W.shape[0]
    fa, fb = _fused_table(tok_table, W, b)
    flat_idx = idx.reshape(-1).astype(jnp.int32)
    out1, outb = _make_gather(V, Bn * Tn)(fa, fb, flat_idx)
    out = _merge_tail(outb, out1)
    return out.reshape(Bn, Tn, V)


# speedup vs baseline: 3.7983x; 2.3658x over previous
import functools

import jax
import jax.numpy as jnp
from jax import lax
from jax.experimental import pallas as pl
from jax.experimental.pallas import tpu as pltpu
from jax.experimental.pallas import tpu_sc as plsc

_NC = 2
_NS = 16
_NW = _NC * _NS

_CHUNK = 128
_NBUF = 4

_BBLK = 512


def _make_gather(V, D, N):
    n_per_w = N // _NW
    n_chunks = n_per_w // _CHUNK
    n_outer = n_chunks // _NBUF
    assert n_chunks % _NBUF == 0 and n_outer >= 2
    mesh = plsc.VectorSubcoreMesh(core_axis_name="c", subcore_axis_name="s")

    @functools.partial(
        pl.kernel,
        out_type=jax.ShapeDtypeStruct((N, D), jnp.float32),
        mesh=mesh,
        scratch_types=[
            pltpu.VMEM((n_per_w,), jnp.int32),
            pltpu.VMEM((_NBUF, _CHUNK, D), jnp.float32),
            pltpu.SemaphoreType.DMA((_NBUF,)),
            pltpu.SemaphoreType.DMA((_NBUF,)),
        ],
        compiler_params=pltpu.CompilerParams(use_tc_tiling_on_sc=False),
    )
    def gather_kernel(table_hbm, idx_hbm, out_hbm, idx_v, rows_v, gsem, osem):
        wid = lax.axis_index("s") * _NC + lax.axis_index("c")
        base = wid * n_per_w
        pltpu.sync_copy(idx_hbm.at[pl.ds(base, n_per_w)], idx_v)

        def g_copy(j, slot):
            return pltpu.make_async_copy(
                table_hbm.at[idx_v.at[pl.ds(j * _CHUNK, _CHUNK)]],
                rows_v.at[slot],
                gsem.at[slot],
            )

        def o_copy(j, slot):
            return pltpu.make_async_copy(
                rows_v.at[slot],
                out_hbm.at[pl.ds(base + j * _CHUNK, _CHUNK)],
                osem.at[slot],
            )

        for s in range(_NBUF - 1):
            g_copy(s, s).start()

        def step(g, s):
            j = g * _NBUF + s
            g_copy(j, s).wait()
            o_copy(j, s).start()
            nxt = (s + _NBUF - 1) % _NBUF
            if s == 0:
                @pl.when(g > 0)
                def _():
                    o_copy(j - 1, nxt).wait()
                g_copy(j + _NBUF - 1, nxt).start()
            else:
                o_copy(j - 1, nxt).wait()

                @pl.when(g < n_outer - 1)
                def _():
                    g_copy(j + _NBUF - 1, nxt).start()

        def body(g, carry):
            for s in range(_NBUF):
                step(g, s)
            return carry

        lax.fori_loop(0, n_outer, body, 0)
        o_copy(n_chunks - 1, (n_chunks - 1) % _NBUF).wait()

    return gather_kernel


def _mm_body(temb_ref, w_ref, b_ref, out_ref):
    ts = temb_ref[0]
    acc = lax.dot_general(
        w_ref[...], ts, (((1,), (1,)), ((), ())),
        preferred_element_type=jnp.float32,
    )
    out_ref[0] = acc + b_ref[...]


def _matmul_slabs(temb, W, b, T, B):
    V, D = W.shape
    temb3 = temb.reshape(T, B, D)
    return pl.pallas_call(
        _mm_body,
        grid=(T, B // _BBLK),
        in_specs=[
            pl.BlockSpec((1, _BBLK, D), lambda t, i: (t, i, 0)),
            pl.BlockSpec((V, D), lambda t, i: (0, 0)),
            pl.BlockSpec((V, 1), lambda t, i: (0, 0)),
        ],
        out_specs=pl.BlockSpec((1, V, _BBLK), lambda t, i: (t, 0, i)),
        out_shape=jax.ShapeDtypeStruct((T, V, B), jnp.float32),
    )(temb3, W, b.reshape(V, 1))


def kernel(idx, tok_table, pos_table, W, b):
    del pos_table
    Bn, Tn = idx.shape
    V, D = W.shape
    idx_t = idx.T.reshape(-1).astype(jnp.int32)
    temb = _make_gather(tok_table.shape[0], D, Bn * Tn)(tok_table, idx_t)
    out_p = _matmul_slabs(temb, W, b, Tn, Bn)
    return jnp.transpose(out_p, (2, 0, 1))


# speedup vs baseline: 4.6703x; 1.2296x over previous
import functools

import jax
import jax.numpy as jnp
from jax import lax
from jax.experimental import pallas as pl
from jax.experimental.pallas import tpu as pltpu
from jax.experimental.pallas import tpu_sc as plsc

_NC = 2
_NS = 16
_NW = _NC * _NS

_CHUNK = 128
_NBUF = 4

_BBLK = 1024


def _make_gather(V, D, N):
    n_per_w = N // _NW
    n_chunks = n_per_w // _CHUNK
    n_outer = n_chunks // _NBUF
    assert n_chunks % _NBUF == 0 and n_outer >= 2
    mesh = plsc.VectorSubcoreMesh(core_axis_name="c", subcore_axis_name="s")

    @functools.partial(
        pl.kernel,
        out_type=jax.ShapeDtypeStruct((N, D), jnp.float32),
        mesh=mesh,
        scratch_types=[
            pltpu.VMEM((n_per_w,), jnp.int32),
            pltpu.VMEM((_NBUF, _CHUNK, D), jnp.float32),
            pltpu.SemaphoreType.DMA((_NBUF,)),
            pltpu.SemaphoreType.DMA((_NBUF,)),
        ],
        compiler_params=pltpu.CompilerParams(use_tc_tiling_on_sc=False),
    )
    def gather_kernel(table_hbm, idx_hbm, out_hbm, idx_v, rows_v, gsem, osem):
        wid = lax.axis_index("s") * _NC + lax.axis_index("c")
        base = wid * n_per_w
        pltpu.sync_copy(idx_hbm.at[pl.ds(base, n_per_w)], idx_v)

        def g_copy(j, slot):
            return pltpu.make_async_copy(
                table_hbm.at[idx_v.at[pl.ds(j * _CHUNK, _CHUNK)]],
                rows_v.at[slot],
                gsem.at[slot],
            )

        def o_copy(j, slot):
            return pltpu.make_async_copy(
                rows_v.at[slot],
                out_hbm.at[pl.ds(base + j * _CHUNK, _CHUNK)],
                osem.at[slot],
            )

        for s in range(_NBUF - 1):
            g_copy(s, s).start()

        def step(g, s):
            j = g * _NBUF + s
            g_copy(j, s).wait()
            o_copy(j, s).start()
            nxt = (s + _NBUF - 1) % _NBUF
            if s == 0:
                @pl.when(g > 0)
                def _():
                    o_copy(j - 1, nxt).wait()
                g_copy(j + _NBUF - 1, nxt).start()
            else:
                o_copy(j - 1, nxt).wait()

                @pl.when(g < n_outer - 1)
                def _():
                    g_copy(j + _NBUF - 1, nxt).start()

        def body(g, carry):
            for s in range(_NBUF):
                step(g, s)
            return carry

        lax.fori_loop(0, n_outer, body, 0)
        o_copy(n_chunks - 1, (n_chunks - 1) % _NBUF).wait()

    return gather_kernel


def _mm_body(temb_ref, w_ref, b_ref, out_ref):
    ts = temb_ref[0]
    acc = lax.dot_general(
        w_ref[...], ts, (((1,), (1,)), ((), ())),
        preferred_element_type=jnp.float32,
    )
    out_ref[0] = acc + b_ref[...]


def _matmul_slabs(temb, W, b, T, B):
    V, D = W.shape
    temb3 = temb.reshape(T, B, D)
    return pl.pallas_call(
        _mm_body,
        grid=(T, B // _BBLK),
        in_specs=[
            pl.BlockSpec((1, _BBLK, D), lambda t, i: (t, i, 0)),
            pl.BlockSpec((V, D), lambda t, i: (0, 0)),
            pl.BlockSpec((V, 1), lambda t, i: (0, 0)),
        ],
        out_specs=pl.BlockSpec((1, V, _BBLK), lambda t, i: (t, 0, i)),
        out_shape=jax.ShapeDtypeStruct((T, V, B), jnp.float32),
    )(temb3, W, b.reshape(V, 1))


def kernel(idx, tok_table, pos_table, W, b):
    del pos_table
    Bn, Tn = idx.shape
    V, D = W.shape
    idx_t = idx.T.reshape(-1).astype(jnp.int32)
    temb = _make_gather(tok_table.shape[0], D, Bn * Tn)(tok_table, idx_t)
    out_p = _matmul_slabs(temb, W, b, Tn, Bn)
    return jnp.transpose(out_p, (2, 0, 1))


# speedup vs baseline: 5.1609x; 1.1050x over previous
import functools

import jax
import jax.numpy as jnp
from jax import lax
from jax.experimental import pallas as pl
from jax.experimental.pallas import tpu as pltpu
from jax.experimental.pallas import tpu_sc as plsc

_NC = 2
_NS = 16
_NW = _NC * _NS

_CHUNK = 128
_NBUF = 4

_BBLK = 2048


def _make_gather(V, D, N):
    n_per_w = N // _NW
    n_chunks = n_per_w // _CHUNK
    n_outer = n_chunks // _NBUF
    assert n_chunks % _NBUF == 0 and n_outer >= 2
    mesh = plsc.VectorSubcoreMesh(core_axis_name="c", subcore_axis_name="s")

    @functools.partial(
        pl.kernel,
        out_type=jax.ShapeDtypeStruct((N, D), jnp.float32),
        mesh=mesh,
        scratch_types=[
            pltpu.VMEM((n_per_w,), jnp.int32),
            pltpu.VMEM((_NBUF, _CHUNK, D), jnp.float32),
            pltpu.SemaphoreType.DMA((_NBUF,)),
            pltpu.SemaphoreType.DMA((_NBUF,)),
        ],
        compiler_params=pltpu.CompilerParams(use_tc_tiling_on_sc=False),
    )
    def gather_kernel(table_hbm, idx_hbm, out_hbm, idx_v, rows_v, gsem, osem):
        wid = lax.axis_index("s") * _NC + lax.axis_index("c")
        base = wid * n_per_w
        pltpu.sync_copy(idx_hbm.at[pl.ds(base, n_per_w)], idx_v)

        def g_copy(j, slot):
            return pltpu.make_async_copy(
                table_hbm.at[idx_v.at[pl.ds(j * _CHUNK, _CHUNK)]],
                rows_v.at[slot],
                gsem.at[slot],
            )

        def o_copy(j, slot):
            return pltpu.make_async_copy(
                rows_v.at[slot],
                out_hbm.at[pl.ds(base + j * _CHUNK, _CHUNK)],
                osem.at[slot],
            )

        for s in range(_NBUF - 1):
            g_copy(s, s).start()

        def step(g, s):
            j = g * _NBUF + s
            g_copy(j, s).wait()
            o_copy(j, s).start()
            nxt = (s + _NBUF - 1) % _NBUF
            if s == 0:
                @pl.when(g > 0)
                def _():
                    o_copy(j - 1, nxt).wait()
                g_copy(j + _NBUF - 1, nxt).start()
            else:
                o_copy(j - 1, nxt).wait()

                @pl.when(g < n_outer - 1)
                def _():
                    g_copy(j + _NBUF - 1, nxt).start()

        def body(g, carry):
            for s in range(_NBUF):
                step(g, s)
            return carry

        lax.fori_loop(0, n_outer, body, 0)
        o_copy(n_chunks - 1, (n_chunks - 1) % _NBUF).wait()

    return gather_kernel


def _mm_body(temb_ref, w_ref, b_ref, out_ref):
    ts = temb_ref[0]
    acc = lax.dot_general(
        w_ref[...], ts, (((1,), (1,)), ((), ())),
        preferred_element_type=jnp.float32,
    )
    out_ref[0] = acc + b_ref[...]


def _matmul_slabs(temb, W, b, T, B):
    V, D = W.shape
    temb3 = temb.reshape(T, B, D)
    return pl.pallas_call(
        _mm_body,
        grid=(T, B // _BBLK),
        in_specs=[
            pl.BlockSpec((1, _BBLK, D), lambda t, i: (t, i, 0)),
            pl.BlockSpec((V, D), lambda t, i: (0, 0)),
            pl.BlockSpec((V, 1), lambda t, i: (0, 0)),
        ],
        out_specs=pl.BlockSpec((1, V, _BBLK), lambda t, i: (t, 0, i)),
        out_shape=jax.ShapeDtypeStruct((T, V, B), jnp.float32),
    )(temb3, W, b.reshape(V, 1))


def kernel(idx, tok_table, pos_table, W, b):
    del pos_table
    Bn, Tn = idx.shape
    V, D = W.shape
    idx_t = idx.T.reshape(-1).astype(jnp.int32)
    temb = _make_gather(tok_table.shape[0], D, Bn * Tn)(tok_table, idx_t)
    out_p = _matmul_slabs(temb, W, b, Tn, Bn)
    return jnp.transpose(out_p, (2, 0, 1))


# speedup vs baseline: 5.2336x; 1.0141x over previous
import functools

import jax
import jax.numpy as jnp
from jax import lax
from jax.experimental import pallas as pl
from jax.experimental.pallas import tpu as pltpu
from jax.experimental.pallas import tpu_sc as plsc

_NC = 2
_NS = 16
_NW = _NC * _NS

_CHUNK = 128
_NBUF = 4

_BBLK = 4096


def _make_gather(V, D, N):
    n_per_w = N // _NW
    n_chunks = n_per_w // _CHUNK
    n_outer = n_chunks // _NBUF
    assert n_chunks % _NBUF == 0 and n_outer >= 2
    mesh = plsc.VectorSubcoreMesh(core_axis_name="c", subcore_axis_name="s")

    @functools.partial(
        pl.kernel,
        out_type=jax.ShapeDtypeStruct((N, D), jnp.float32),
        mesh=mesh,
        scratch_types=[
            pltpu.VMEM((n_per_w,), jnp.int32),
            pltpu.VMEM((_NBUF, _CHUNK, D), jnp.float32),
            pltpu.SemaphoreType.DMA((_NBUF,)),
            pltpu.SemaphoreType.DMA((_NBUF,)),
        ],
        compiler_params=pltpu.CompilerParams(use_tc_tiling_on_sc=False),
    )
    def gather_kernel(table_hbm, idx_hbm, out_hbm, idx_v, rows_v, gsem, osem):
        wid = lax.axis_index("s") * _NC + lax.axis_index("c")
        base = wid * n_per_w
        pltpu.sync_copy(idx_hbm.at[pl.ds(base, n_per_w)], idx_v)

        def g_copy(j, slot):
            return pltpu.make_async_copy(
                table_hbm.at[idx_v.at[pl.ds(j * _CHUNK, _CHUNK)]],
                rows_v.at[slot],
                gsem.at[slot],
            )

        def o_copy(j, slot):
            return pltpu.make_async_copy(
                rows_v.at[slot],
                out_hbm.at[pl.ds(base + j * _CHUNK, _CHUNK)],
                osem.at[slot],
            )

        for s in range(_NBUF - 1):
            g_copy(s, s).start()

        def step(g, s):
            j = g * _NBUF + s
            g_copy(j, s).wait()
            o_copy(j, s).start()
            nxt = (s + _NBUF - 1) % _NBUF
            if s == 0:
                @pl.when(g > 0)
                def _():
                    o_copy(j - 1, nxt).wait()
                g_copy(j + _NBUF - 1, nxt).start()
            else:
                o_copy(j - 1, nxt).wait()

                @pl.when(g < n_outer - 1)
                def _():
                    g_copy(j + _NBUF - 1, nxt).start()

        def body(g, carry):
            for s in range(_NBUF):
                step(g, s)
            return carry

        lax.fori_loop(0, n_outer, body, 0)
        o_copy(n_chunks - 1, (n_chunks - 1) % _NBUF).wait()

    return gather_kernel


def _mm_body(temb_ref, w_ref, b_ref, out_ref):
    ts = temb_ref[0]
    acc = lax.dot_general(
        w_ref[...], ts, (((1,), (1,)), ((), ())),
        preferred_element_type=jnp.float32,
    )
    out_ref[0] = acc + b_ref[...]


def _matmul_slabs(temb, W, b, T, B):
    V, D = W.shape
    temb3 = temb.reshape(T, B, D)
    return pl.pallas_call(
        _mm_body,
        grid=(T, B // _BBLK),
        in_specs=[
            pl.BlockSpec((1, _BBLK, D), lambda t, i: (t, i, 0)),
            pl.BlockSpec((V, D), lambda t, i: (0, 0)),
            pl.BlockSpec((V, 1), lambda t, i: (0, 0)),
        ],
        out_specs=pl.BlockSpec((1, V, _BBLK), lambda t, i: (t, 0, i)),
        out_shape=jax.ShapeDtypeStruct((T, V, B), jnp.float32),
    )(temb3, W, b.reshape(V, 1))


def kernel(idx, tok_table, pos_table, W, b):
    del pos_table
    Bn, Tn = idx.shape
    V, D = W.shape
    idx_t = idx.T.reshape(-1).astype(jnp.int32)
    temb = _make_gather(tok_table.shape[0], D, Bn * Tn)(tok_table, idx_t)
    out_p = _matmul_slabs(temb, W, b, Tn, Bn)
    return jnp.transpose(out_p, (2, 0, 1))
